# baseline (speedup 1.0000x reference)
import functools

import numpy as np
import jax
import jax.numpy as jnp
from jax import lax
from jax.experimental import pallas as pl
from jax.experimental.pallas import tpu as pltpu
from jax.experimental.pallas import tpu_sc as plsc

_B = 4096
_D = 32
_S = 25
_N = 100000
_NC = 2
_NS = 16
_NW = _NC * _NS
_L = 16
_NVQ = _B // _L
_UNROLL = 8

_PERM = np.asarray(
    [31, 7, 4, 29, 16, 19, 2, 5, 30, 3, 22, 6, 18, 10, 11, 15,
     20, 8, 24, 9, 25, 13, 14, 17, 23, 0, 21, 26, 1, 28, 27, 12],
    dtype=np.int32,
)
_CTRL_LEN = _L + _D + _L + (_NW - _S)

_mesh = plsc.VectorSubcoreMesh(core_axis_name="c", subcore_axis_name="s")


@functools.partial(
    pl.kernel,
    out_type=jax.ShapeDtypeStruct((_S, _B), jnp.int32),
    mesh=_mesh,
    compiler_params=pltpu.CompilerParams(
        needs_layout_passes=False,
        disable_bounds_checks=True,
        disable_semaphore_checks=True,
        skip_device_barrier=True,
    ),
    scratch_types=[
        pltpu.VMEM((_CTRL_LEN,), jnp.int32),
        pltpu.VMEM((_B,), jnp.int32),
        pltpu.VMEM((_N,), jnp.int32),
        pltpu.VMEM((_B,), jnp.int32),
        pltpu.SemaphoreType.DMA,
        pltpu.SemaphoreType.DMA,
    ],
)
def _sample_sc(tableT_hbm, ids_hbm, ctrl_hbm, out_hbm,
               ctrl_v, ids_v, row_v, gat_v, sem_ids, sem_row):
    wid = lax.axis_index("s") * _NC + lax.axis_index("c")

    @pl.when(wid < _S)
    def _():
        a_ids = pltpu.async_copy(ids_hbm, ids_v, sem_ids)
        pltpu.sync_copy(ctrl_hbm, ctrl_v)
        sv = ctrl_v[pl.ds(0, _L)]
        pv = ctrl_v[pl.ds(_L + sv[0] - _S + wid, _L)]
        r = sv[1] * _D + pv[0]
        pltpu.async_copy(tableT_hbm.at[r], row_v, sem_row).wait()
        a_ids.wait()

        def body(v, carry):
            base = v * (_UNROLL * _L)
            for u in range(_UNROLL):
                ivec = ids_v[pl.ds(base + u * _L, _L)]
                gat_v[pl.ds(base + u * _L, _L)] = plsc.load_gather(row_v, [ivec])
            return carry

        lax.fori_loop(0, _NVQ // _UNROLL, body, 0)
        pltpu.sync_copy(gat_v, out_hbm.at[wid])


def kernel(ids, num_samples, t, adj_info):
    T, N, D = adj_info.shape
    tableT = jnp.swapaxes(adj_info, 1, 2).reshape(T * D, N)
    ctrl = (
        jnp.zeros((_CTRL_LEN,), jnp.int32)
        .at[0].set(num_samples)
        .at[1].set(t)
        .at[_L : _L + _D].set(jnp.asarray(_PERM))
    )
    out = _sample_sc(tableT, ids, ctrl)
    return out.T
